# baseline (speedup 1.0000x reference)
import functools

import jax
import jax.numpy as jnp
from jax import lax
from jax.experimental import pallas as pl
from jax.experimental.pallas import tpu as pltpu
from jax.experimental.pallas import tpu_sc as plsc

B = 16384
BSC = 8192
BTC = B - BSC
GIDS = 16
ED = 64
LANES = 128
SLAB = 128
NC, NS = 2, 16
NW = NC * NS
BPW = BSC // NW
NBUF = 4
CHUNK = 16
HALF = 128

_sc_mesh = plsc.VectorSubcoreMesh(core_axis_name="c", subcore_axis_name="s")


@functools.partial(
    pl.kernel,
    mesh=_sc_mesh,
    out_type=[
        jax.ShapeDtypeStruct((ED, BSC), jnp.float32),
        jax.ShapeDtypeStruct((ED, BSC), jnp.float32),
    ],
    scratch_types=[
        pltpu.VMEM((BPW,), jnp.int32),
        pltpu.VMEM((BPW,), jnp.int32),
        pltpu.VMEM((NBUF, ED, SLAB), jnp.float32),
        pltpu.VMEM((NBUF, ED, SLAB), jnp.float32),
        pltpu.VMEM((ED, HALF), jnp.float32),
        pltpu.VMEM((ED, HALF), jnp.float32),
        pltpu.SemaphoreType.DMA((NBUF,)),
        pltpu.SemaphoreType.DMA((NBUF,)),
    ],
    compiler_params=pltpu.CompilerParams(needs_layout_passes=False),
)
def _gather_sc(uid_hbm, iid_hbm, utabT_hbm, itabT_hbm, uoutT_hbm, ioutT_hbm,
               uid_v, iid_v, uslab, islab, uout_v, iout_v, usem, isem):
    wid = lax.axis_index("s") * NC + lax.axis_index("c")
    base = wid * BPW
    pltpu.sync_copy(uid_hbm.at[pl.ds(base, BPW)], uid_v)
    pltpu.sync_copy(iid_hbm.at[pl.ds(base, BPW)], iid_v)

    rows16 = lax.iota(jnp.int32, 16)

    def fire(u, v, b):
        ut = pl.multiple_of((u >> 7) * SLAB, SLAB)
        pltpu.async_copy(utabT_hbm.at[:, pl.ds(ut, SLAB)], uslab.at[b],
                         usem.at[b])
        it = pl.multiple_of((v >> 7) * SLAB, SLAB)
        pltpu.async_copy(itabT_hbm.at[:, pl.ds(it, SLAB)], islab.at[b],
                         isem.at[b])

    def extract(u, v, col_i, b):
        pltpu.make_async_copy(utabT_hbm.at[:, pl.ds(0, SLAB)], uslab.at[b],
                              usem.at[b]).wait()
        pltpu.make_async_copy(itabT_hbm.at[:, pl.ds(0, SLAB)], islab.at[b],
                              isem.at[b]).wait()
        ul = jnp.full((16,), u & (SLAB - 1), jnp.int32)
        il = jnp.full((16,), v & (SLAB - 1), jnp.int32)
        col = jnp.full((16,), col_i, jnp.int32)
        for c in range(ED // 16):
            r = rows16 + (16 * c)
            uvec = plsc.load_gather(uslab.at[b], [r, ul])
            plsc.store_scatter(uout_v, [r, col], uvec)
            ivec = plsc.load_gather(islab.at[b], [r, il])
            plsc.store_scatter(iout_v, [r, col], ivec)

    for h in range(BPW // HALF):
        h0 = h * HALF

        @pl.loop(h0, h0 + HALF, step=CHUNK)
        def _chunk(o):
            uvec = uid_v[pl.ds(o, CHUNK)]
            ivec = iid_v[pl.ds(o, CHUNK)]

            for j in range(CHUNK):
                i = o + j
                b = j % NBUF
                pj = (j - NBUF) % CHUNK

                @pl.when(i - h0 >= NBUF)
                def _():
                    po = o if j >= NBUF else o - CHUNK
                    puvec = uid_v[pl.ds(po, CHUNK)]
                    pivec = iid_v[pl.ds(po, CHUNK)]
                    extract(puvec[pj], pivec[pj], (po + pj) - h0, b)

                fire(uvec[j], ivec[j], b)

        last = h0 + HALF - CHUNK
        luvec = uid_v[pl.ds(last, CHUNK)]
        livec = iid_v[pl.ds(last, CHUNK)]
        for j in range(NBUF):
            pj = CHUNK - NBUF + j
            extract(luvec[pj], livec[pj], HALF - NBUF + j, pj % NBUF)

        pltpu.sync_copy(uout_v, uoutT_hbm.at[:, pl.ds(base + h0, HALF)])
        pltpu.sync_copy(iout_v, ioutT_hbm.at[:, pl.ds(base + h0, HALF)])


def _gather_tc_body(uid_ref, iid_ref, *refs):
    uslabs = refs[:GIDS]
    islabs = refs[GIDS:2 * GIDS]
    uo_ref, io_ref = refs[2 * GIDS], refs[2 * GIDS + 1]
    i = pl.program_id(0)
    lanes = lax.broadcasted_iota(jnp.int32, (LANES, 1), 0)
    ucols, icols = [], []
    for j in range(GIDS):
        ul = uid_ref[i * GIDS + j] & (LANES - 1)
        um = (lanes == ul).astype(jnp.float32)
        ucols.append(jnp.dot(uslabs[j][...], um,
                             preferred_element_type=jnp.float32))
        il = iid_ref[i * GIDS + j] & (LANES - 1)
        im = (lanes == il).astype(jnp.float32)
        icols.append(jnp.dot(islabs[j][...], im,
                             preferred_element_type=jnp.float32))
    uo_ref[...] = jnp.concatenate(ucols, axis=1).T
    io_ref[...] = jnp.concatenate(icols, axis=1).T


def _gather_tc(uid, iid, utabT, itabT):
    def uspec(j):
        return pl.BlockSpec(
            (ED, LANES), lambda i, u, v, j=j: (0, u[i * GIDS + j] >> 7))

    def ispec(j):
        return pl.BlockSpec(
            (ED, LANES), lambda i, u, v, j=j: (0, v[i * GIDS + j] >> 7))

    grid_spec = pltpu.PrefetchScalarGridSpec(
        num_scalar_prefetch=2,
        grid=(BTC // GIDS,),
        in_specs=[uspec(j) for j in range(GIDS)]
                 + [ispec(j) for j in range(GIDS)],
        out_specs=[pl.BlockSpec((GIDS, ED), lambda i, u, v: (i, 0))] * 2,
    )
    return pl.pallas_call(
        _gather_tc_body,
        grid_spec=grid_spec,
        out_shape=[jax.ShapeDtypeStruct((BTC, ED), jnp.float32)] * 2,
    )(uid, iid, *([utabT] * GIDS), *([itabT] * GIDS))


BLK = 2048


def _mlp_body(emb_nt, featT_ref, ueT_ref, ieT_ref, w1t_ref, b1_ref,
              w2t_ref, b2_ref, w3ut_ref, w3it_ref, w3ft_ref, b3_ref,
              w4t_ref, b4_ref, w5t_ref, b5_ref, out_ref):
    dot = lambda a, b: jnp.dot(a, b, preferred_element_type=jnp.float32)
    if emb_nt:
        nt = lambda a, b: lax.dot_general(
            a, b, (((1,), (1,)), ((), ())),
            preferred_element_type=jnp.float32)
    else:
        nt = dot
    h = jnp.maximum(dot(w1t_ref[...], featT_ref[...]) + b1_ref[...], 0.0)
    f = jnp.maximum(dot(w2t_ref[...], h) + b2_ref[...], 0.0)
    y = (nt(w3ut_ref[...], ueT_ref[...])
         + nt(w3it_ref[...], ieT_ref[...])
         + dot(w3ft_ref[...], f)
         + b3_ref[...])
    y = jnp.maximum(y, 0.0)
    y = jnp.maximum(dot(w4t_ref[...], y) + b4_ref[...], 0.0)
    z = dot(w5t_ref[...], y) + b5_ref[...]
    out_ref[...] = 1.0 / (1.0 + jnp.exp(-z))


def _mlp_tc(n, emb_nt, featT, ueT, ieT, W1T, b1, W2T, b2, W3uT, W3iT, W3fT,
            b3, W4T, b4, W5T, b5):
    nblk = n // BLK
    col_spec = lambda h: pl.BlockSpec((h, BLK), lambda i: (0, i))
    row_spec = lambda w: pl.BlockSpec((BLK, w), lambda i: (i, 0))
    emb_spec = row_spec(ED) if emb_nt else col_spec(ED)
    full = lambda a: pl.BlockSpec(a.shape, lambda i: (0,) * a.ndim)
    return pl.pallas_call(
        functools.partial(_mlp_body, emb_nt),
        grid=(nblk,),
        in_specs=[
            col_spec(featT.shape[0]),
            emb_spec,
            emb_spec,
            full(W1T), full(b1), full(W2T), full(b2),
            full(W3uT), full(W3iT), full(W3fT), full(b3),
            full(W4T), full(b4), full(W5T), full(b5),
        ],
        out_specs=pl.BlockSpec((1, BLK), lambda i: (0, i)),
        out_shape=jax.ShapeDtypeStruct((1, n), jnp.float32),
    )(featT, ueT, ieT, W1T, b1, W2T, b2, W3uT, W3iT, W3fT, b3,
      W4T, b4, W5T, b5)


def kernel(user_ids, item_ids, features, user_table, item_table,
           W1, b1, W2, b2, W3, b3, W4, b4, W5, b5):
    uid = user_ids.astype(jnp.int32)
    iid = item_ids.astype(jnp.int32)
    utabT = user_table.T
    itabT = item_table.T
    featT = features.T
    ueT_sc, ieT_sc = _gather_sc(uid[:BSC], iid[:BSC], utabT, itabT)
    ueT_tc, ieT_tc = _gather_tc(uid[BSC:], iid[BSC:], utabT, itabT)
    weights = (W1.T, b1.reshape(-1, 1), W2.T, b2.reshape(-1, 1),
               W3[:ED].T, W3[ED:2 * ED].T, W3[2 * ED:].T, b3.reshape(-1, 1),
               W4.T, b4.reshape(-1, 1), W5.T, b5.reshape(-1, 1))
    out_sc = _mlp_tc(BSC, False, featT[:, :BSC], ueT_sc, ieT_sc, *weights)
    out_tc = _mlp_tc(BTC, True, featT[:, BSC:], ueT_tc, ieT_tc, *weights)
    return jnp.concatenate([out_sc, out_tc], axis=1).reshape(B)


# speedup vs baseline: 1.7714x; 1.7714x over previous
import functools

import jax
import jax.numpy as jnp
from jax import lax
from jax.experimental import pallas as pl
from jax.experimental.pallas import tpu as pltpu
from jax.experimental.pallas import tpu_sc as plsc

B = 16384
BSC = 12288
BTC = B - BSC
GIDS = 16
ED = 64
LANES = 128
SLAB = 128
NC, NS = 2, 16
NW = NC * NS
BPW = BSC // NW
NBUF = 4
CHUNK = 16
HALF = 128

_sc_mesh = plsc.VectorSubcoreMesh(core_axis_name="c", subcore_axis_name="s")


@functools.partial(
    pl.kernel,
    mesh=_sc_mesh,
    out_type=[
        jax.ShapeDtypeStruct((ED, BSC), jnp.float32),
        jax.ShapeDtypeStruct((ED, BSC), jnp.float32),
    ],
    scratch_types=[
        pltpu.VMEM((BPW,), jnp.int32),
        pltpu.VMEM((BPW,), jnp.int32),
        pltpu.VMEM((NBUF, ED, SLAB), jnp.float32),
        pltpu.VMEM((NBUF, ED, SLAB), jnp.float32),
        pltpu.VMEM((ED, HALF), jnp.float32),
        pltpu.VMEM((ED, HALF), jnp.float32),
        pltpu.SemaphoreType.DMA((NBUF,)),
        pltpu.SemaphoreType.DMA((NBUF,)),
    ],
    compiler_params=pltpu.CompilerParams(needs_layout_passes=False),
)
def _gather_sc(uid_hbm, iid_hbm, utabT_hbm, itabT_hbm, uoutT_hbm, ioutT_hbm,
               uid_v, iid_v, uslab, islab, uout_v, iout_v, usem, isem):
    wid = lax.axis_index("s") * NC + lax.axis_index("c")
    base = wid * BPW
    pltpu.sync_copy(uid_hbm.at[pl.ds(base, BPW)], uid_v)
    pltpu.sync_copy(iid_hbm.at[pl.ds(base, BPW)], iid_v)

    rows16 = lax.iota(jnp.int32, 16)

    def fire(u, v, b):
        ut = pl.multiple_of((u >> 7) * SLAB, SLAB)
        pltpu.async_copy(utabT_hbm.at[:, pl.ds(ut, SLAB)], uslab.at[b],
                         usem.at[b])
        it = pl.multiple_of((v >> 7) * SLAB, SLAB)
        pltpu.async_copy(itabT_hbm.at[:, pl.ds(it, SLAB)], islab.at[b],
                         isem.at[b])

    def extract(u, v, col_i, b):
        pltpu.make_async_copy(utabT_hbm.at[:, pl.ds(0, SLAB)], uslab.at[b],
                              usem.at[b]).wait()
        pltpu.make_async_copy(itabT_hbm.at[:, pl.ds(0, SLAB)], islab.at[b],
                              isem.at[b]).wait()
        ul = jnp.full((16,), u & (SLAB - 1), jnp.int32)
        il = jnp.full((16,), v & (SLAB - 1), jnp.int32)
        col = jnp.full((16,), col_i, jnp.int32)
        for c in range(ED // 16):
            r = rows16 + (16 * c)
            uvec = plsc.load_gather(uslab.at[b], [r, ul])
            plsc.store_scatter(uout_v, [r, col], uvec)
            ivec = plsc.load_gather(islab.at[b], [r, il])
            plsc.store_scatter(iout_v, [r, col], ivec)

    for h in range(BPW // HALF):
        h0 = h * HALF

        @pl.loop(h0, h0 + HALF, step=CHUNK)
        def _chunk(o):
            uvec = uid_v[pl.ds(o, CHUNK)]
            ivec = iid_v[pl.ds(o, CHUNK)]

            for j in range(CHUNK):
                i = o + j
                b = j % NBUF
                pj = (j - NBUF) % CHUNK

                @pl.when(i - h0 >= NBUF)
                def _():
                    po = o if j >= NBUF else o - CHUNK
                    puvec = uid_v[pl.ds(po, CHUNK)]
                    pivec = iid_v[pl.ds(po, CHUNK)]
                    extract(puvec[pj], pivec[pj], (po + pj) - h0, b)

                fire(uvec[j], ivec[j], b)

        last = h0 + HALF - CHUNK
        luvec = uid_v[pl.ds(last, CHUNK)]
        livec = iid_v[pl.ds(last, CHUNK)]
        for j in range(NBUF):
            pj = CHUNK - NBUF + j
            extract(luvec[pj], livec[pj], HALF - NBUF + j, pj % NBUF)

        pltpu.sync_copy(uout_v, uoutT_hbm.at[:, pl.ds(base + h0, HALF)])
        pltpu.sync_copy(iout_v, ioutT_hbm.at[:, pl.ds(base + h0, HALF)])


def _gather_tc_body(uid_ref, iid_ref, *refs):
    uslabs = refs[:GIDS]
    islabs = refs[GIDS:2 * GIDS]
    uo_ref, io_ref = refs[2 * GIDS], refs[2 * GIDS + 1]
    i = pl.program_id(0)
    lanes = lax.broadcasted_iota(jnp.int32, (1, LANES), 1)
    nt = lambda a, b: lax.dot_general(
        a, b, (((1,), (1,)), ((), ())), preferred_element_type=jnp.float32)
    urows, irows = [], []
    for j in range(GIDS):
        ul = uid_ref[i * GIDS + j] & (LANES - 1)
        urows.append(nt((lanes == ul).astype(jnp.float32), uslabs[j][...]))
        il = iid_ref[i * GIDS + j] & (LANES - 1)
        irows.append(nt((lanes == il).astype(jnp.float32), islabs[j][...]))
    uo_ref[...] = jnp.concatenate(urows, axis=0)
    io_ref[...] = jnp.concatenate(irows, axis=0)


def _gather_tc(uid, iid, utabT, itabT):
    def uspec(j):
        return pl.BlockSpec(
            (ED, LANES), lambda i, u, v, j=j: (0, u[i * GIDS + j] >> 7))

    def ispec(j):
        return pl.BlockSpec(
            (ED, LANES), lambda i, u, v, j=j: (0, v[i * GIDS + j] >> 7))

    grid_spec = pltpu.PrefetchScalarGridSpec(
        num_scalar_prefetch=2,
        grid=(BTC // GIDS,),
        in_specs=[uspec(j) for j in range(GIDS)]
                 + [ispec(j) for j in range(GIDS)],
        out_specs=[pl.BlockSpec((GIDS, ED), lambda i, u, v: (i, 0))] * 2,
    )
    return pl.pallas_call(
        _gather_tc_body,
        grid_spec=grid_spec,
        out_shape=[jax.ShapeDtypeStruct((BTC, ED), jnp.float32)] * 2,
    )(uid, iid, *([utabT] * GIDS), *([itabT] * GIDS))


BLK = 2048


def _mlp_body(emb_nt, featT_ref, ueT_ref, ieT_ref, w1t_ref, b1_ref,
              w2t_ref, b2_ref, w3ut_ref, w3it_ref, w3ft_ref, b3_ref,
              w4t_ref, b4_ref, w5t_ref, b5_ref, out_ref):
    dot = lambda a, b: jnp.dot(a, b, preferred_element_type=jnp.float32)
    if emb_nt:
        nt = lambda a, b: lax.dot_general(
            a, b, (((1,), (1,)), ((), ())),
            preferred_element_type=jnp.float32)
    else:
        nt = dot
    h = jnp.maximum(dot(w1t_ref[...], featT_ref[...]) + b1_ref[...], 0.0)
    f = jnp.maximum(dot(w2t_ref[...], h) + b2_ref[...], 0.0)
    y = (nt(w3ut_ref[...], ueT_ref[...])
         + nt(w3it_ref[...], ieT_ref[...])
         + dot(w3ft_ref[...], f)
         + b3_ref[...])
    y = jnp.maximum(y, 0.0)
    y = jnp.maximum(dot(w4t_ref[...], y) + b4_ref[...], 0.0)
    z = dot(w5t_ref[...], y) + b5_ref[...]
    out_ref[...] = 1.0 / (1.0 + jnp.exp(-z))


def _mlp_tc(n, emb_nt, featT, ueT, ieT, W1T, b1, W2T, b2, W3uT, W3iT, W3fT,
            b3, W4T, b4, W5T, b5):
    nblk = n // BLK
    col_spec = lambda h: pl.BlockSpec((h, BLK), lambda i: (0, i))
    row_spec = lambda w: pl.BlockSpec((BLK, w), lambda i: (i, 0))
    emb_spec = row_spec(ED) if emb_nt else col_spec(ED)
    full = lambda a: pl.BlockSpec(a.shape, lambda i: (0,) * a.ndim)
    return pl.pallas_call(
        functools.partial(_mlp_body, emb_nt),
        grid=(nblk,),
        in_specs=[
            col_spec(featT.shape[0]),
            emb_spec,
            emb_spec,
            full(W1T), full(b1), full(W2T), full(b2),
            full(W3uT), full(W3iT), full(W3fT), full(b3),
            full(W4T), full(b4), full(W5T), full(b5),
        ],
        out_specs=pl.BlockSpec((1, BLK), lambda i: (0, i)),
        out_shape=jax.ShapeDtypeStruct((1, n), jnp.float32),
    )(featT, ueT, ieT, W1T, b1, W2T, b2, W3uT, W3iT, W3fT, b3,
      W4T, b4, W5T, b5)


def kernel(user_ids, item_ids, features, user_table, item_table,
           W1, b1, W2, b2, W3, b3, W4, b4, W5, b5):
    uid = user_ids.astype(jnp.int32)
    iid = item_ids.astype(jnp.int32)
    utabT = user_table.T
    itabT = item_table.T
    featT = features.T
    ueT_sc, ieT_sc = _gather_sc(uid[:BSC], iid[:BSC], utabT, itabT)
    ueT_tc, ieT_tc = _gather_tc(uid[BSC:], iid[BSC:], utabT, itabT)
    weights = (W1.T, b1.reshape(-1, 1), W2.T, b2.reshape(-1, 1),
               W3[:ED].T, W3[ED:2 * ED].T, W3[2 * ED:].T, b3.reshape(-1, 1),
               W4.T, b4.reshape(-1, 1), W5.T, b5.reshape(-1, 1))
    out_sc = _mlp_tc(BSC, False, featT[:, :BSC], ueT_sc, ieT_sc, *weights)
    out_tc = _mlp_tc(BTC, True, featT[:, BSC:], ueT_tc, ieT_tc, *weights)
    return jnp.concatenate([out_sc, out_tc], axis=1).reshape(B)


# speedup vs baseline: 1.7797x; 1.0047x over previous
import functools

import jax
import jax.numpy as jnp
from jax import lax
from jax.experimental import pallas as pl
from jax.experimental.pallas import tpu as pltpu
from jax.experimental.pallas import tpu_sc as plsc

B = 16384
BSC = 12288
BTC = B - BSC
GIDS = 16
ED = 64
LANES = 128
SLAB = 128
NC, NS = 2, 16
NW = NC * NS
BPW = BSC // NW
NBUF = 4
CHUNK = 16
HALF = 384

_sc_mesh = plsc.VectorSubcoreMesh(core_axis_name="c", subcore_axis_name="s")


@functools.partial(
    pl.kernel,
    mesh=_sc_mesh,
    out_type=[
        jax.ShapeDtypeStruct((ED, BSC), jnp.float32),
        jax.ShapeDtypeStruct((ED, BSC), jnp.float32),
    ],
    scratch_types=[
        pltpu.VMEM((BPW,), jnp.int32),
        pltpu.VMEM((BPW,), jnp.int32),
        pltpu.VMEM((NBUF, ED, SLAB), jnp.float32),
        pltpu.VMEM((NBUF, ED, SLAB), jnp.float32),
        pltpu.VMEM((ED, HALF), jnp.float32),
        pltpu.VMEM((ED, HALF), jnp.float32),
        pltpu.SemaphoreType.DMA((NBUF,)),
        pltpu.SemaphoreType.DMA((NBUF,)),
    ],
    compiler_params=pltpu.CompilerParams(needs_layout_passes=False),
)
def _gather_sc(uid_hbm, iid_hbm, utabT_hbm, itabT_hbm, uoutT_hbm, ioutT_hbm,
               uid_v, iid_v, uslab, islab, uout_v, iout_v, usem, isem):
    wid = lax.axis_index("s") * NC + lax.axis_index("c")
    base = wid * BPW
    pltpu.sync_copy(uid_hbm.at[pl.ds(base, BPW)], uid_v)
    pltpu.sync_copy(iid_hbm.at[pl.ds(base, BPW)], iid_v)

    rows16 = lax.iota(jnp.int32, 16)

    def fire(u, v, b):
        ut = pl.multiple_of((u >> 7) * SLAB, SLAB)
        pltpu.async_copy(utabT_hbm.at[:, pl.ds(ut, SLAB)], uslab.at[b],
                         usem.at[b])
        it = pl.multiple_of((v >> 7) * SLAB, SLAB)
        pltpu.async_copy(itabT_hbm.at[:, pl.ds(it, SLAB)], islab.at[b],
                         isem.at[b])

    def extract(u, v, col_i, b):
        pltpu.make_async_copy(utabT_hbm.at[:, pl.ds(0, SLAB)], uslab.at[b],
                              usem.at[b]).wait()
        pltpu.make_async_copy(itabT_hbm.at[:, pl.ds(0, SLAB)], islab.at[b],
                              isem.at[b]).wait()
        ul = jnp.full((16,), u & (SLAB - 1), jnp.int32)
        il = jnp.full((16,), v & (SLAB - 1), jnp.int32)
        col = jnp.full((16,), col_i, jnp.int32)
        for c in range(ED // 16):
            r = rows16 + (16 * c)
            uvec = plsc.load_gather(uslab.at[b], [r, ul])
            plsc.store_scatter(uout_v, [r, col], uvec)
            ivec = plsc.load_gather(islab.at[b], [r, il])
            plsc.store_scatter(iout_v, [r, col], ivec)

    for h in range(BPW // HALF):
        h0 = h * HALF

        @pl.loop(h0, h0 + HALF, step=CHUNK)
        def _chunk(o):
            uvec = uid_v[pl.ds(o, CHUNK)]
            ivec = iid_v[pl.ds(o, CHUNK)]

            for j in range(CHUNK):
                i = o + j
                b = j % NBUF
                pj = (j - NBUF) % CHUNK

                @pl.when(i - h0 >= NBUF)
                def _():
                    po = o if j >= NBUF else o - CHUNK
                    puvec = uid_v[pl.ds(po, CHUNK)]
                    pivec = iid_v[pl.ds(po, CHUNK)]
                    extract(puvec[pj], pivec[pj], (po + pj) - h0, b)

                fire(uvec[j], ivec[j], b)

        last = h0 + HALF - CHUNK
        luvec = uid_v[pl.ds(last, CHUNK)]
        livec = iid_v[pl.ds(last, CHUNK)]
        for j in range(NBUF):
            pj = CHUNK - NBUF + j
            extract(luvec[pj], livec[pj], HALF - NBUF + j, pj % NBUF)

        pltpu.sync_copy(uout_v, uoutT_hbm.at[:, pl.ds(base + h0, HALF)])
        pltpu.sync_copy(iout_v, ioutT_hbm.at[:, pl.ds(base + h0, HALF)])


def _gather_tc_body(uid_ref, iid_ref, *refs):
    uslabs = refs[:GIDS]
    islabs = refs[GIDS:2 * GIDS]
    uo_ref, io_ref = refs[2 * GIDS], refs[2 * GIDS + 1]
    i = pl.program_id(0)
    lanes = lax.broadcasted_iota(jnp.int32, (1, LANES), 1)
    nt = lambda a, b: lax.dot_general(
        a, b, (((1,), (1,)), ((), ())), preferred_element_type=jnp.float32)
    urows, irows = [], []
    for j in range(GIDS):
        ul = uid_ref[i * GIDS + j] & (LANES - 1)
        urows.append(nt((lanes == ul).astype(jnp.float32), uslabs[j][...]))
        il = iid_ref[i * GIDS + j] & (LANES - 1)
        irows.append(nt((lanes == il).astype(jnp.float32), islabs[j][...]))
    uo_ref[...] = jnp.concatenate(urows, axis=0)
    io_ref[...] = jnp.concatenate(irows, axis=0)


def _gather_tc(uid, iid, utabT, itabT):
    def uspec(j):
        return pl.BlockSpec(
            (ED, LANES), lambda i, u, v, j=j: (0, u[i * GIDS + j] >> 7))

    def ispec(j):
        return pl.BlockSpec(
            (ED, LANES), lambda i, u, v, j=j: (0, v[i * GIDS + j] >> 7))

    grid_spec = pltpu.PrefetchScalarGridSpec(
        num_scalar_prefetch=2,
        grid=(BTC // GIDS,),
        in_specs=[uspec(j) for j in range(GIDS)]
                 + [ispec(j) for j in range(GIDS)],
        out_specs=[pl.BlockSpec((GIDS, ED), lambda i, u, v: (i, 0))] * 2,
    )
    return pl.pallas_call(
        _gather_tc_body,
        grid_spec=grid_spec,
        out_shape=[jax.ShapeDtypeStruct((BTC, ED), jnp.float32)] * 2,
    )(uid, iid, *([utabT] * GIDS), *([itabT] * GIDS))


BLK = 2048


def _mlp_body(emb_nt, featT_ref, ueT_ref, ieT_ref, w1t_ref, b1_ref,
              w2t_ref, b2_ref, w3ut_ref, w3it_ref, w3ft_ref, b3_ref,
              w4t_ref, b4_ref, w5t_ref, b5_ref, out_ref):
    dot = lambda a, b: jnp.dot(a, b, preferred_element_type=jnp.float32)
    if emb_nt:
        nt = lambda a, b: lax.dot_general(
            a, b, (((1,), (1,)), ((), ())),
            preferred_element_type=jnp.float32)
    else:
        nt = dot
    h = jnp.maximum(dot(w1t_ref[...], featT_ref[...]) + b1_ref[...], 0.0)
    f = jnp.maximum(dot(w2t_ref[...], h) + b2_ref[...], 0.0)
    y = (nt(w3ut_ref[...], ueT_ref[...])
         + nt(w3it_ref[...], ieT_ref[...])
         + dot(w3ft_ref[...], f)
         + b3_ref[...])
    y = jnp.maximum(y, 0.0)
    y = jnp.maximum(dot(w4t_ref[...], y) + b4_ref[...], 0.0)
    z = dot(w5t_ref[...], y) + b5_ref[...]
    out_ref[...] = 1.0 / (1.0 + jnp.exp(-z))


def _mlp_tc(n, emb_nt, featT, ueT, ieT, W1T, b1, W2T, b2, W3uT, W3iT, W3fT,
            b3, W4T, b4, W5T, b5):
    nblk = n // BLK
    col_spec = lambda h: pl.BlockSpec((h, BLK), lambda i: (0, i))
    row_spec = lambda w: pl.BlockSpec((BLK, w), lambda i: (i, 0))
    emb_spec = row_spec(ED) if emb_nt else col_spec(ED)
    full = lambda a: pl.BlockSpec(a.shape, lambda i: (0,) * a.ndim)
    return pl.pallas_call(
        functools.partial(_mlp_body, emb_nt),
        grid=(nblk,),
        in_specs=[
            col_spec(featT.shape[0]),
            emb_spec,
            emb_spec,
            full(W1T), full(b1), full(W2T), full(b2),
            full(W3uT), full(W3iT), full(W3fT), full(b3),
            full(W4T), full(b4), full(W5T), full(b5),
        ],
        out_specs=pl.BlockSpec((1, BLK), lambda i: (0, i)),
        out_shape=jax.ShapeDtypeStruct((1, n), jnp.float32),
    )(featT, ueT, ieT, W1T, b1, W2T, b2, W3uT, W3iT, W3fT, b3,
      W4T, b4, W5T, b5)


def kernel(user_ids, item_ids, features, user_table, item_table,
           W1, b1, W2, b2, W3, b3, W4, b4, W5, b5):
    uid = user_ids.astype(jnp.int32)
    iid = item_ids.astype(jnp.int32)
    utabT = user_table.T
    itabT = item_table.T
    featT = features.T
    ueT_sc, ieT_sc = _gather_sc(uid[:BSC], iid[:BSC], utabT, itabT)
    ueT_tc, ieT_tc = _gather_tc(uid[BSC:], iid[BSC:], utabT, itabT)
    weights = (W1.T, b1.reshape(-1, 1), W2.T, b2.reshape(-1, 1),
               W3[:ED].T, W3[ED:2 * ED].T, W3[2 * ED:].T, b3.reshape(-1, 1),
               W4.T, b4.reshape(-1, 1), W5.T, b5.reshape(-1, 1))
    out_sc = _mlp_tc(BSC, False, featT[:, :BSC], ueT_sc, ieT_sc, *weights)
    out_tc = _mlp_tc(BTC, True, featT[:, BSC:], ueT_tc, ieT_tc, *weights)
    return jnp.concatenate([out_sc, out_tc], axis=1).reshape(B)
